# baseline (speedup 1.0000x reference)
import jax
import jax.numpy as jnp
from jax.experimental import pallas as pl
from jax.experimental.pallas import tpu as pltpu

T = 20
B = 8
NMAX = 128
N = B * NMAX
IN_DIM = 128
HID = 64
OUT_DIM = 64


def _gcn_body(adj_ref, x_ref, m_ref, W1_ref, b1_ref, W2_ref, b2_ref, seq_ref):
    adj_b = adj_ref[0].astype(jnp.bfloat16)
    m = m_ref[0]

    deg = jnp.dot(m.astype(jnp.bfloat16), adj_b,
                  preferred_element_type=jnp.float32)
    deg = deg * m + m
    dinv = jax.lax.rsqrt(jnp.maximum(deg, 1.0))
    mdinv = m * dinv
    selfw = dinv * dinv * m

    h0T = jax.lax.dot_general(W1_ref[...], x_ref[0], (((0,), (1,)), ((), ())),
                              preferred_element_type=jnp.float32)
    agg1 = jnp.dot((h0T * mdinv).astype(jnp.bfloat16), adj_b,
                   preferred_element_type=jnp.float32)
    h1T = jnp.maximum(mdinv * agg1 + selfw * h0T + b1_ref[...], 0.0)

    h2pT = jax.lax.dot_general(W2_ref[...], h1T, (((0,), (0,)), ((), ())),
                               preferred_element_type=jnp.float32)
    agg2 = jnp.dot((h2pT * mdinv).astype(jnp.bfloat16), adj_b,
                   preferred_element_type=jnp.float32)
    seqT = m * (mdinv * agg2 + selfw * h2pT + b2_ref[...])
    seq_ref[0] = seqT.astype(jnp.bfloat16)


def _lstm_body(seq_ref, mask_ref, Wih_ref, Whh_ref, bg_ref, Wfc_ref, bfc_ref,
               out_ref, h_s, c_s, hn_s):
    h_s[...] = jnp.zeros_like(h_s)
    c_s[...] = jnp.zeros_like(c_s)
    hn_s[...] = jnp.zeros_like(hn_s)
    lengths = jnp.maximum(jnp.sum(mask_ref[...], axis=0, keepdims=True), 1.0)
    last_t = lengths - 1.0
    Wih = Wih_ref[...].astype(jnp.bfloat16)
    Whh = Whh_ref[...]

    def step(t, carry):
        seq_t = seq_ref[t]
        gates = (jnp.dot(Wih, seq_t, preferred_element_type=jnp.float32)
                 + jnp.dot(Whh, h_s[...], preferred_element_type=jnp.float32)
                 + bg_ref[...])
        gi = jax.nn.sigmoid(gates[0 * HID:1 * HID])
        gf = jax.nn.sigmoid(gates[1 * HID:2 * HID])
        gg = jnp.tanh(gates[2 * HID:3 * HID])
        go = jax.nn.sigmoid(gates[3 * HID:4 * HID])
        c = gf * c_s[...] + gi * gg
        h = go * jnp.tanh(c)
        c_s[...] = c
        h_s[...] = h
        sel = last_t == t.astype(jnp.float32)
        hn_s[...] = jnp.where(sel, h, hn_s[...])
        return carry

    jax.lax.fori_loop(0, T, step, 0)
    out_ref[...] = (jax.lax.dot_general(
        Wfc_ref[...], hn_s[...], (((0,), (0,)), ((), ())),
        preferred_element_type=jnp.float32) + bfc_ref[...])


def kernel(x, big_batch_adjacency, ego_mask, W_gcn1, b_gcn1, W_gcn2, b_gcn2,
           W_ih, W_hh, b_ih, b_hh, W_fc, b_fc):
    mask_flat = jnp.transpose(ego_mask, (1, 0, 2)).reshape(T, N).astype(jnp.float32)
    mask3 = mask_flat[:, None, :]
    b_gates = (b_ih + b_hh).reshape(4 * HID, 1)

    seq = pl.pallas_call(
        _gcn_body,
        grid=(T,),
        in_specs=[
            pl.BlockSpec((1, N, N), lambda t: (t, 0, 0)),
            pl.BlockSpec((1, N, IN_DIM), lambda t: (t, 0, 0)),
            pl.BlockSpec((1, 1, N), lambda t: (t, 0, 0)),
            pl.BlockSpec((IN_DIM, HID), lambda t: (0, 0)),
            pl.BlockSpec((HID, 1), lambda t: (0, 0)),
            pl.BlockSpec((HID, HID), lambda t: (0, 0)),
            pl.BlockSpec((HID, 1), lambda t: (0, 0)),
        ],
        out_specs=pl.BlockSpec((1, HID, N), lambda t: (t, 0, 0)),
        out_shape=jax.ShapeDtypeStruct((T, HID, N), jnp.bfloat16),
        compiler_params=pltpu.CompilerParams(
            dimension_semantics=("arbitrary",),
        ),
    )(big_batch_adjacency, x, mask3,
      W_gcn1, b_gcn1.reshape(HID, 1), W_gcn2, b_gcn2.reshape(HID, 1))

    outT = pl.pallas_call(
        _lstm_body,
        grid=(1,),
        in_specs=[
            pl.BlockSpec((T, HID, N), lambda i: (0, 0, 0)),
            pl.BlockSpec((T, N), lambda i: (0, 0)),
            pl.BlockSpec((4 * HID, HID), lambda i: (0, 0)),
            pl.BlockSpec((4 * HID, HID), lambda i: (0, 0)),
            pl.BlockSpec((4 * HID, 1), lambda i: (0, 0)),
            pl.BlockSpec((HID, OUT_DIM), lambda i: (0, 0)),
            pl.BlockSpec((OUT_DIM, 1), lambda i: (0, 0)),
        ],
        out_specs=pl.BlockSpec((OUT_DIM, N), lambda i: (0, 0)),
        out_shape=jax.ShapeDtypeStruct((OUT_DIM, N), jnp.float32),
        scratch_shapes=[
            pltpu.VMEM((HID, N), jnp.float32),
            pltpu.VMEM((HID, N), jnp.float32),
            pltpu.VMEM((HID, N), jnp.float32),
        ],
    )(seq, mask_flat, W_ih, W_hh, b_gates, W_fc, b_fc.reshape(OUT_DIM, 1))

    return outT.T.reshape(B, NMAX, OUT_DIM)


# speedup vs baseline: 1.2410x; 1.2410x over previous
import jax
import jax.numpy as jnp
from jax.experimental import pallas as pl
from jax.experimental.pallas import tpu as pltpu

T = 20
B = 8
NMAX = 128
N = B * NMAX
IN_DIM = 128
HID = 64
OUT_DIM = 64


TPB = 2


def _fused_body(adj_ref, x_ref, m_ref, mask_ref,
                W1_ref, b1_ref, W2_ref, b2_ref,
                Wih_ref, Whh_ref, bg_ref, Wfc_ref, bfc_ref,
                out_ref, h_s, c_s, hn_s):
    k = pl.program_id(0)

    @pl.when(k == 0)
    def _init():
        h_s[...] = jnp.zeros_like(h_s)
        c_s[...] = jnp.zeros_like(c_s)
        hn_s[...] = jnp.zeros_like(hn_s)

    for j in range(TPB):
        _one_step(k * TPB + j, j, adj_ref, x_ref, m_ref, mask_ref,
                  W1_ref, b1_ref, W2_ref, b2_ref,
                  Wih_ref, Whh_ref, bg_ref, Wfc_ref, bfc_ref,
                  out_ref, h_s, c_s, hn_s)


def _one_step(t, j, adj_ref, x_ref, m_ref, mask_ref,
              W1_ref, b1_ref, W2_ref, b2_ref,
              Wih_ref, Whh_ref, bg_ref, Wfc_ref, bfc_ref,
              out_ref, h_s, c_s, hn_s):
    adj_b = adj_ref[j].astype(jnp.bfloat16)
    m = m_ref[j]

    deg = jnp.dot(m.astype(jnp.bfloat16), adj_b,
                  preferred_element_type=jnp.float32)
    deg = deg * m + m
    dinv = jax.lax.rsqrt(jnp.maximum(deg, 1.0))
    mdinv = m * dinv
    selfw = dinv * dinv * m

    h0T = jax.lax.dot_general(W1_ref[...], x_ref[j], (((0,), (1,)), ((), ())),
                              preferred_element_type=jnp.float32)
    agg1 = jnp.dot((h0T * mdinv).astype(jnp.bfloat16), adj_b,
                   preferred_element_type=jnp.float32)
    h1T = jnp.maximum(mdinv * agg1 + selfw * h0T + b1_ref[...], 0.0)

    h2pT = jax.lax.dot_general(W2_ref[...], h1T, (((0,), (0,)), ((), ())),
                               preferred_element_type=jnp.float32)
    agg2 = jnp.dot((h2pT * mdinv).astype(jnp.bfloat16), adj_b,
                   preferred_element_type=jnp.float32)
    seqT = m * (mdinv * agg2 + selfw * h2pT + b2_ref[...])

    gatesT = (jnp.dot(Wih_ref[...], seqT, preferred_element_type=jnp.float32)
              + jnp.dot(Whh_ref[...], h_s[...], preferred_element_type=jnp.float32)
              + bg_ref[...])
    gi = jax.nn.sigmoid(gatesT[0 * HID:1 * HID])
    gf = jax.nn.sigmoid(gatesT[1 * HID:2 * HID])
    gg = jnp.tanh(gatesT[2 * HID:3 * HID])
    go = jax.nn.sigmoid(gatesT[3 * HID:4 * HID])
    c = gf * c_s[...] + gi * gg
    h = go * jnp.tanh(c)
    c_s[...] = c
    h_s[...] = h

    lengths = jnp.maximum(jnp.sum(mask_ref[...], axis=0, keepdims=True), 1.0)
    sel = (lengths - 1.0) == jnp.float32(1.0) * t
    hn = jnp.where(sel, h, hn_s[...])
    hn_s[...] = hn

    @pl.when(t == T - 1)
    def _fin():
        out_ref[...] = (jax.lax.dot_general(
            Wfc_ref[...], hn, (((0,), (0,)), ((), ())),
            preferred_element_type=jnp.float32) + bfc_ref[...])


def kernel(x, big_batch_adjacency, ego_mask, W_gcn1, b_gcn1, W_gcn2, b_gcn2,
           W_ih, W_hh, b_ih, b_hh, W_fc, b_fc):
    mask_flat = jnp.transpose(ego_mask, (1, 0, 2)).reshape(T, N).astype(jnp.float32)
    mask3 = mask_flat[:, None, :]
    b_gates = (b_ih + b_hh).reshape(4 * HID, 1)

    outT = pl.pallas_call(
        _fused_body,
        grid=(T // TPB,),
        in_specs=[
            pl.BlockSpec((TPB, N, N), lambda t: (t, 0, 0)),
            pl.BlockSpec((TPB, N, IN_DIM), lambda t: (t, 0, 0)),
            pl.BlockSpec((TPB, 1, N), lambda t: (t, 0, 0)),
            pl.BlockSpec((T, N), lambda t: (0, 0)),
            pl.BlockSpec((IN_DIM, HID), lambda t: (0, 0)),
            pl.BlockSpec((HID, 1), lambda t: (0, 0)),
            pl.BlockSpec((HID, HID), lambda t: (0, 0)),
            pl.BlockSpec((HID, 1), lambda t: (0, 0)),
            pl.BlockSpec((4 * HID, HID), lambda t: (0, 0)),
            pl.BlockSpec((4 * HID, HID), lambda t: (0, 0)),
            pl.BlockSpec((4 * HID, 1), lambda t: (0, 0)),
            pl.BlockSpec((HID, OUT_DIM), lambda t: (0, 0)),
            pl.BlockSpec((OUT_DIM, 1), lambda t: (0, 0)),
        ],
        out_specs=pl.BlockSpec((OUT_DIM, N), lambda t: (0, 0)),
        out_shape=jax.ShapeDtypeStruct((OUT_DIM, N), jnp.float32),
        scratch_shapes=[
            pltpu.VMEM((HID, N), jnp.float32),
            pltpu.VMEM((HID, N), jnp.float32),
            pltpu.VMEM((HID, N), jnp.float32),
        ],
        compiler_params=pltpu.CompilerParams(
            dimension_semantics=("arbitrary",),
        ),
    )(big_batch_adjacency, x, mask3, mask_flat,
      W_gcn1, b_gcn1.reshape(HID, 1), W_gcn2, b_gcn2.reshape(HID, 1),
      W_ih, W_hh, b_gates, W_fc, b_fc.reshape(OUT_DIM, 1))

    return outT.T.reshape(B, NMAX, OUT_DIM)
